# baseline (speedup 1.0000x reference)
import functools

import jax
import jax.numpy as jnp
from jax.experimental import pallas as pl
from jax.experimental.pallas import tpu as pltpu


def _norm_body(scale, t_ref, g_ref, o_ref):
    t = t_ref[...]
    ss = jnp.sum(t * t, axis=1, keepdims=True)
    inv = jax.lax.rsqrt(jnp.maximum(ss, 1e-24))
    o_ref[...] = (t * (inv * scale) * g_ref[...]).astype(jnp.bfloat16)


def _rmsnorm_bf16(t2, gamma, scale):
    r, d = t2.shape
    br = 512 if r % 512 == 0 else r
    return pl.pallas_call(
        functools.partial(_norm_body, scale),
        grid=(r // br,),
        in_specs=[
            pl.BlockSpec((br, d), lambda i: (i, 0)),
            pl.BlockSpec((1, d), lambda i: (0, 0)),
        ],
        out_specs=pl.BlockSpec((br, d), lambda i: (i, 0)),
        out_shape=jax.ShapeDtypeStruct((r, d), jnp.bfloat16),
    )(t2, gamma.reshape(1, d))


def _dispatch_body(bk, scale, xn_ref, se_ref, g_ref, slots_ref, l_ref):
    xn = xn_ref[0]
    sef = se_ref[...]
    ss = jnp.sum(sef * sef, axis=1, keepdims=True)
    inv = jax.lax.rsqrt(jnp.maximum(ss, 1e-24))
    se = (sef * (inv * scale) * g_ref[...]).astype(jnp.bfloat16)
    logits = jax.lax.dot_general(
        xn, se, (((1,), (1,)), ((), ())),
        preferred_element_type=jnp.float32)
    l_ref[0] = logits
    cmax = jnp.max(logits, axis=0, keepdims=True)
    pf = jnp.exp(logits - cmax)
    p = pf.astype(jnp.bfloat16)
    csum = jnp.sum(pf, axis=0, keepdims=True)
    s = jax.lax.dot_general(
        p, xn, (((0,), (0,)), ((), ())),
        preferred_element_type=jnp.float32)
    s = s * (1.0 / csum).reshape(bk, 1)
    slots_ref[0] = s.astype(jnp.bfloat16)


def _ffn_body(bsz, s, d, nq, dq, slots_ref, w1_ref, b1_ref, w2_ref, b2_ref, y_ref):
    ht = pl.program_id(1)
    a = slots_ref[...].reshape(bsz * s, d)
    acc = None
    for q in range(nq):
        w1_q = w1_ref[0][:, q * dq:(q + 1) * dq].astype(jnp.bfloat16)
        h = jax.lax.dot_general(
            a, w1_q, (((1,), (0,)), ((), ())),
            preferred_element_type=jnp.float32) + b1_ref[0][:, q * dq:(q + 1) * dq]
        g = 0.5 * h * (1.0 + jax.lax.erf(h * 0.7071067811865476))
        w2_q = w2_ref[0][q * dq:(q + 1) * dq, :].astype(jnp.bfloat16)
        part = jax.lax.dot_general(
            g.astype(jnp.bfloat16), w2_q,
            (((1,), (0,)), ((), ())), preferred_element_type=jnp.float32)
        acc = part if acc is None else acc + part
    res = acc.astype(jnp.bfloat16).reshape(bsz, s, d)

    @pl.when(ht == 0)
    def _():
        y_ref[...] = (acc + b2_ref[0]).astype(jnp.bfloat16).reshape(bsz, s, d)

    @pl.when(ht != 0)
    def _():
        y_ref[...] = y_ref[...] + res


def _combine_body(bn, nsplit, l_ref, y_ref, out_ref):
    y = y_ref[0]
    hw = bn // nsplit
    for h in range(nsplit):
        logits = l_ref[0, h * hw:(h + 1) * hw, :]
        rmax = jnp.max(logits, axis=1, keepdims=True)
        p = jnp.exp(logits - rmax)
        rsum = jnp.sum(p, axis=1, keepdims=True)
        o = jax.lax.dot_general(
            p.astype(jnp.bfloat16), y, (((1,), (0,)), ((), ())),
            preferred_element_type=jnp.float32)
        out_ref[0, h * hw:(h + 1) * hw, :] = o * (1.0 / rsum)


def kernel(x, norm_gamma, slot_norm_gamma, slot_embeds, w1, b1, w2, b2):
    bsz, n, d = x.shape
    e, s, _ = slot_embeds.shape
    k = e * s
    dh = w1.shape[2]
    scale = float(d) ** 0.5
    bk = 512 if k % 512 == 0 else k
    kt = k // bk
    bnc = 256 if n % 256 == 0 else n
    ntc = n // bnc
    dhb = 2048 if dh % 2048 == 0 else dh
    nht = dh // dhb
    dq = 1024 if dhb % 1024 == 0 else dhb
    nq = dhb // dq

    xn = _rmsnorm_bf16(x.reshape(bsz * n, d), norm_gamma, scale).reshape(bsz, n, d)

    slots = pl.pallas_call(
        functools.partial(_dispatch_body, bk, scale),
        grid=(bsz, kt),
        in_specs=[
            pl.BlockSpec((1, n, d), lambda b, i: (b, 0, 0)),
            pl.BlockSpec((bk, d), lambda b, i: (i, 0)),
            pl.BlockSpec((1, d), lambda b, i: (0, 0)),
        ],
        out_specs=[
            pl.BlockSpec((1, bk, d), lambda b, i: (b, i, 0)),
            pl.BlockSpec((1, n, bk), lambda b, i: (b, 0, i)),
        ],
        out_shape=[
            jax.ShapeDtypeStruct((bsz, k, d), jnp.bfloat16),
            jax.ShapeDtypeStruct((bsz, n, k), jnp.float32),
        ],
        compiler_params=pltpu.CompilerParams(
            dimension_semantics=("parallel", "parallel")),
    )(xn, slot_embeds.reshape(k, d), slot_norm_gamma.reshape(1, d))
    slots, lg = slots

    y = pl.pallas_call(
        functools.partial(_ffn_body, bsz, s, d, nq, dq),
        grid=(e, nht),
        in_specs=[
            pl.BlockSpec((bsz, s, d), lambda i, h: (0, i, 0)),
            pl.BlockSpec((1, d, dhb), lambda i, h: (i, 0, h)),
            pl.BlockSpec((1, 1, dhb), lambda i, h: (i, 0, h)),
            pl.BlockSpec((1, dhb, d), lambda i, h: (i, h, 0)),
            pl.BlockSpec((1, 1, d), lambda i, h: (i, 0, 0)),
        ],
        out_specs=pl.BlockSpec((bsz, s, d), lambda i, h: (0, i, 0)),
        out_shape=jax.ShapeDtypeStruct((bsz, k, d), jnp.bfloat16),
        compiler_params=pltpu.CompilerParams(
            dimension_semantics=("parallel", "arbitrary")),
    )(slots, w1, b1.reshape(e, 1, dh), w2, b2.reshape(e, 1, d))

    out = pl.pallas_call(
        functools.partial(_combine_body, bnc, 1),
        grid=(bsz, ntc),
        in_specs=[
            pl.BlockSpec((1, bnc, k), lambda b, i: (b, i, 0)),
            pl.BlockSpec((1, k, d), lambda b, i: (b, 0, 0)),
        ],
        out_specs=pl.BlockSpec((1, bnc, d), lambda b, i: (b, i, 0)),
        out_shape=jax.ShapeDtypeStruct((bsz, n, d), jnp.float32),
        compiler_params=pltpu.CompilerParams(
            dimension_semantics=("parallel", "parallel")),
    )(lg, y)
    return out


# speedup vs baseline: 1.0110x; 1.0110x over previous
import functools

import jax
import jax.numpy as jnp
from jax.experimental import pallas as pl
from jax.experimental.pallas import tpu as pltpu


def _norm_body(scale, t_ref, g_ref, o_ref):
    t = t_ref[...]
    ss = jnp.sum(t * t, axis=1, keepdims=True)
    inv = jax.lax.rsqrt(jnp.maximum(ss, 1e-24))
    o_ref[...] = (t * (inv * scale) * g_ref[...]).astype(jnp.bfloat16)


def _rmsnorm_bf16(t2, gamma, scale):
    r, d = t2.shape
    br = 512 if r % 512 == 0 else r
    return pl.pallas_call(
        functools.partial(_norm_body, scale),
        grid=(r // br,),
        in_specs=[
            pl.BlockSpec((br, d), lambda i: (i, 0)),
            pl.BlockSpec((1, d), lambda i: (0, 0)),
        ],
        out_specs=pl.BlockSpec((br, d), lambda i: (i, 0)),
        out_shape=jax.ShapeDtypeStruct((r, d), jnp.bfloat16),
    )(t2, gamma.reshape(1, d))


def _dispatch_body(bk, scale, xn_ref, se_ref, g_ref, slots_ref, l_ref):
    xn = xn_ref[0]
    sef = se_ref[...]
    ss = jnp.sum(sef * sef, axis=1, keepdims=True)
    inv = jax.lax.rsqrt(jnp.maximum(ss, 1e-24))
    se = (sef * (inv * scale) * g_ref[...]).astype(jnp.bfloat16)
    logits = jax.lax.dot_general(
        xn, se, (((1,), (1,)), ((), ())),
        preferred_element_type=jnp.float32)
    l_ref[0] = logits
    cmax = jnp.max(logits, axis=0, keepdims=True)
    pf = jnp.exp(logits - cmax)
    p = pf.astype(jnp.bfloat16)
    csum = jnp.sum(pf, axis=0, keepdims=True)
    s = jax.lax.dot_general(
        p, xn, (((0,), (0,)), ((), ())),
        preferred_element_type=jnp.float32)
    s = s * (1.0 / csum).reshape(bk, 1)
    slots_ref[0] = s.astype(jnp.bfloat16)


def _ffn_body(bsz, s, d, nq, dq, slots_ref, w1_ref, b1_ref, w2_ref, b2_ref, y_ref):
    ht = pl.program_id(1)
    a = slots_ref[...].reshape(bsz * s, d)
    acc = None
    for q in range(nq):
        w1_q = w1_ref[0][:, q * dq:(q + 1) * dq].astype(jnp.bfloat16)
        h = jax.lax.dot_general(
            a, w1_q, (((1,), (0,)), ((), ())),
            preferred_element_type=jnp.float32) + b1_ref[0][:, q * dq:(q + 1) * dq]
        g = 0.5 * h * (1.0 + jax.lax.erf(h * 0.7071067811865476))
        w2_q = w2_ref[0][q * dq:(q + 1) * dq, :].astype(jnp.bfloat16)
        part = jax.lax.dot_general(
            g.astype(jnp.bfloat16), w2_q,
            (((1,), (0,)), ((), ())), preferred_element_type=jnp.float32)
        acc = part if acc is None else acc + part
    res = acc.astype(jnp.bfloat16).reshape(bsz, s, d)

    @pl.when(ht == 0)
    def _():
        y_ref[...] = (acc + b2_ref[0]).astype(jnp.bfloat16).reshape(bsz, s, d)

    @pl.when(ht != 0)
    def _():
        y_ref[...] = y_ref[...] + res


def _combine_body(bn, nsplit, l_ref, y_ref, out_ref):
    y = y_ref[0]
    hw = bn // nsplit
    for h in range(nsplit):
        logits = l_ref[0, h * hw:(h + 1) * hw, :]
        rmax = jnp.max(logits, axis=1, keepdims=True)
        p = jnp.exp(logits - rmax)
        rsum = jnp.sum(p, axis=1, keepdims=True)
        o = jax.lax.dot_general(
            p.astype(jnp.bfloat16), y, (((1,), (0,)), ((), ())),
            preferred_element_type=jnp.float32)
        out_ref[0, h * hw:(h + 1) * hw, :] = o * (1.0 / rsum)


def kernel(x, norm_gamma, slot_norm_gamma, slot_embeds, w1, b1, w2, b2):
    bsz, n, d = x.shape
    e, s, _ = slot_embeds.shape
    k = e * s
    dh = w1.shape[2]
    scale = float(d) ** 0.5
    bk = 512 if k % 512 == 0 else k
    kt = k // bk
    bnc = 512 if n % 512 == 0 else n
    ntc = n // bnc
    dhb = 2048 if dh % 2048 == 0 else dh
    nht = dh // dhb
    dq = 1024 if dhb % 1024 == 0 else dhb
    nq = dhb // dq

    xn = _rmsnorm_bf16(x.reshape(bsz * n, d), norm_gamma, scale).reshape(bsz, n, d)

    slots = pl.pallas_call(
        functools.partial(_dispatch_body, bk, scale),
        grid=(bsz, kt),
        in_specs=[
            pl.BlockSpec((1, n, d), lambda b, i: (b, 0, 0)),
            pl.BlockSpec((bk, d), lambda b, i: (i, 0)),
            pl.BlockSpec((1, d), lambda b, i: (0, 0)),
        ],
        out_specs=[
            pl.BlockSpec((1, bk, d), lambda b, i: (b, i, 0)),
            pl.BlockSpec((1, n, bk), lambda b, i: (b, 0, i)),
        ],
        out_shape=[
            jax.ShapeDtypeStruct((bsz, k, d), jnp.bfloat16),
            jax.ShapeDtypeStruct((bsz, n, k), jnp.float32),
        ],
        compiler_params=pltpu.CompilerParams(
            dimension_semantics=("parallel", "parallel")),
    )(xn, slot_embeds.reshape(k, d), slot_norm_gamma.reshape(1, d))
    slots, lg = slots

    y = pl.pallas_call(
        functools.partial(_ffn_body, bsz, s, d, nq, dq),
        grid=(e, nht),
        in_specs=[
            pl.BlockSpec((bsz, s, d), lambda i, h: (0, i, 0)),
            pl.BlockSpec((1, d, dhb), lambda i, h: (i, 0, h)),
            pl.BlockSpec((1, 1, dhb), lambda i, h: (i, 0, h)),
            pl.BlockSpec((1, dhb, d), lambda i, h: (i, h, 0)),
            pl.BlockSpec((1, 1, d), lambda i, h: (i, 0, 0)),
        ],
        out_specs=pl.BlockSpec((bsz, s, d), lambda i, h: (0, i, 0)),
        out_shape=jax.ShapeDtypeStruct((bsz, k, d), jnp.bfloat16),
        compiler_params=pltpu.CompilerParams(
            dimension_semantics=("parallel", "arbitrary")),
    )(slots, w1, b1.reshape(e, 1, dh), w2, b2.reshape(e, 1, d))

    out = pl.pallas_call(
        functools.partial(_combine_body, bnc, 1),
        grid=(bsz, ntc),
        in_specs=[
            pl.BlockSpec((1, bnc, k), lambda b, i: (b, i, 0)),
            pl.BlockSpec((1, k, d), lambda b, i: (b, 0, 0)),
        ],
        out_specs=pl.BlockSpec((1, bnc, d), lambda b, i: (b, i, 0)),
        out_shape=jax.ShapeDtypeStruct((bsz, n, d), jnp.float32),
        compiler_params=pltpu.CompilerParams(
            dimension_semantics=("parallel", "parallel")),
    )(lg, y)
    return out
